# baseline (speedup 1.0000x reference)
import functools

import jax
import jax.numpy as jnp
from jax import lax
from jax.experimental import pallas as pl
from jax.experimental.pallas import tpu as pltpu
from jax.experimental.pallas import tpu_sc as plsc

D = 64
NC = 2
NS = 16
NW = NC * NS
C = 1024


@functools.partial(jax.jit, static_argnums=(2,))
def _gather(idx_flat, table, B):
    b_per_w = B // NW
    n_chunks = b_per_w // C
    mesh = plsc.VectorSubcoreMesh(core_axis_name="c", subcore_axis_name="s")

    @functools.partial(
        pl.kernel,
        mesh=mesh,
        out_type=jax.ShapeDtypeStruct((B, D), jnp.float32),
        compiler_params=pltpu.CompilerParams(use_tc_tiling_on_sc=False),
        scratch_types=[
            pltpu.VMEM((C,), jnp.int32),
            pltpu.VMEM((C, D), jnp.float32),
            pltpu.SemaphoreType.DMA,
        ],
    )
    def k(idx_hbm, table_hbm, out_hbm, idx_v, rows_v, sem):
        wid = lax.axis_index("s") * NC + lax.axis_index("c")
        base = wid * b_per_w

        def body(i, carry):
            off = base + i * C
            pltpu.sync_copy(idx_hbm.at[pl.ds(off, C)], idx_v)
            pltpu.async_copy(table_hbm.at[idx_v], rows_v, sem).wait()
            pltpu.sync_copy(rows_v, out_hbm.at[pl.ds(off, C)])
            return carry

        lax.fori_loop(0, n_chunks, body, 0)

    return k(idx_flat, table)


def kernel(x, table):
    b, s = x.shape
    flat = x.reshape(b * s).astype(jnp.int32)
    out = _gather(flat, table, b * s)
    return out.reshape(b, s, D)


# speedup vs baseline: 1.0020x; 1.0020x over previous
import functools

import jax
import jax.numpy as jnp
from jax import lax
from jax.experimental import pallas as pl
from jax.experimental.pallas import tpu as pltpu
from jax.experimental.pallas import tpu_sc as plsc

D = 64
NC = 2
NS = 16
NW = NC * NS
C = 512


@functools.partial(jax.jit, static_argnums=(2,))
def _gather(idx_flat, table, B):
    b_per_w = B // NW
    n_chunks = b_per_w // C
    assert n_chunks % 2 == 0 and n_chunks >= 4
    mesh = plsc.VectorSubcoreMesh(core_axis_name="c", subcore_axis_name="s")

    @functools.partial(
        pl.kernel,
        mesh=mesh,
        out_type=jax.ShapeDtypeStruct((B, D), jnp.float32),
        compiler_params=pltpu.CompilerParams(use_tc_tiling_on_sc=False),
        scratch_types=[
            pltpu.VMEM((C,), jnp.int32),
            pltpu.VMEM((C,), jnp.int32),
            pltpu.VMEM((C, D), jnp.float32),
            pltpu.VMEM((C, D), jnp.float32),
            pltpu.SemaphoreType.DMA,
            pltpu.SemaphoreType.DMA,
            pltpu.SemaphoreType.DMA,
            pltpu.SemaphoreType.DMA,
        ],
    )
    def k(idx_hbm, table_hbm, out_hbm, iv0, iv1, rows0, rows1,
          sg0, sg1, sw0, sw1):
        wid = lax.axis_index("s") * NC + lax.axis_index("c")
        base = wid * b_per_w
        iv = (iv0, iv1)
        rows = (rows0, rows1)
        sg = (sg0, sg1)
        sw = (sw0, sw1)

        def stage_idx(i, b):
            pltpu.sync_copy(idx_hbm.at[pl.ds(base + i * C, C)], iv[b])

        def start_gather(b):
            pltpu.async_copy(table_hbm.at[iv[b]], rows[b], sg[b])

        def wait_gather(b):
            pltpu.make_async_copy(table_hbm.at[iv[b]], rows[b], sg[b]).wait()

        def start_write(i, b):
            pltpu.async_copy(rows[b], out_hbm.at[pl.ds(base + i * C, C)], sw[b])

        def wait_write(b):
            pltpu.make_async_copy(rows[b], out_hbm.at[pl.ds(base, C)], sw[b]).wait()

        stage_idx(0, 0)
        start_gather(0)

        wait_gather(0)
        start_write(0, 0)
        stage_idx(1, 1)
        start_gather(1)

        def body(j, carry):
            i = 2 * j + 1
            wait_gather(1)
            start_write(i, 1)
            wait_write(0)
            stage_idx(i + 1, 0)
            start_gather(0)
            wait_gather(0)
            start_write(i + 1, 0)
            wait_write(1)
            stage_idx(i + 2, 1)
            start_gather(1)
            return carry

        lax.fori_loop(0, (n_chunks - 2) // 2, body, 0)

        wait_gather(1)
        start_write(n_chunks - 1, 1)
        wait_write(0)
        wait_write(1)

    return k(idx_flat, table)


def kernel(x, table):
    b, s = x.shape
    flat = x.reshape(b * s).astype(jnp.int32)
    out = _gather(flat, table, b * s)
    return out.reshape(b, s, D)


# speedup vs baseline: 1.0059x; 1.0039x over previous
import functools

import jax
import jax.numpy as jnp
from jax import lax
from jax.experimental import pallas as pl
from jax.experimental.pallas import tpu as pltpu
from jax.experimental.pallas import tpu_sc as plsc

D = 64
NC = 2
NS = 16
NW = NC * NS
C = 512


def _make(BATCH, SEQ):
    assert SEQ == C
    q_per_w = BATCH // NW
    assert q_per_w % 2 == 0 and q_per_w >= 4
    mesh = plsc.VectorSubcoreMesh(core_axis_name="c", subcore_axis_name="s")

    @functools.partial(
        pl.kernel,
        mesh=mesh,
        out_type=jax.ShapeDtypeStruct((BATCH, SEQ, D), jnp.float32),
        compiler_params=pltpu.CompilerParams(use_tc_tiling_on_sc=False),
        scratch_types=[
            pltpu.VMEM((C,), jnp.int32),
            pltpu.VMEM((C,), jnp.int32),
            pltpu.VMEM((C, D), jnp.float32),
            pltpu.VMEM((C, D), jnp.float32),
            pltpu.SemaphoreType.DMA,
            pltpu.SemaphoreType.DMA,
            pltpu.SemaphoreType.DMA,
            pltpu.SemaphoreType.DMA,
        ],
    )
    def k(x_hbm, table_hbm, out_hbm, iv0, iv1, rows0, rows1,
          sg0, sg1, sw0, sw1):
        wid = lax.axis_index("s") * NC + lax.axis_index("c")
        base = wid * q_per_w
        iv = (iv0, iv1)
        rows = (rows0, rows1)
        sg = (sg0, sg1)
        sw = (sw0, sw1)

        def stage_idx(q, b):
            pltpu.sync_copy(x_hbm.at[base + q], iv[b])

        def start_gather(b):
            pltpu.async_copy(table_hbm.at[iv[b]], rows[b], sg[b])

        def wait_gather(b):
            pltpu.make_async_copy(table_hbm.at[iv[b]], rows[b], sg[b]).wait()

        def start_write(q, b):
            pltpu.async_copy(rows[b], out_hbm.at[base + q], sw[b])

        def wait_write(b):
            pltpu.make_async_copy(rows[b], out_hbm.at[base], sw[b]).wait()

        stage_idx(0, 0)
        start_gather(0)

        wait_gather(0)
        start_write(0, 0)
        stage_idx(1, 1)
        start_gather(1)

        def body(j, carry):
            q = 2 * j + 1
            wait_gather(1)
            start_write(q, 1)
            wait_write(0)
            stage_idx(q + 1, 0)
            start_gather(0)
            wait_gather(0)
            start_write(q + 1, 0)
            wait_write(1)
            stage_idx(q + 2, 1)
            start_gather(1)
            return carry

        lax.fori_loop(0, (q_per_w - 2) // 2, body, 0)

        wait_gather(1)
        start_write(q_per_w - 1, 1)
        wait_write(0)
        wait_write(1)

    return k


def kernel(x, table):
    b, s = x.shape
    return _make(b, s)(x.astype(jnp.int32), table)
